# baseline (speedup 1.0000x reference)
import functools

import jax
import jax.numpy as jnp
from jax import lax
from jax.experimental import pallas as pl
from jax.experimental.pallas import tpu as pltpu
from jax.experimental.pallas import tpu_sc as plsc

_L = 16


@functools.lru_cache(maxsize=None)
def _build_gather(V: int, D: int, B: int):
    info = plsc.get_sparse_core_info()
    NC, NS = info.num_cores, info.num_subcores
    NW = NC * NS
    W = 2 * D
    assert B % NW == 0 and D % _L == 0 and V % 2 == 0
    b_per_w = B // NW
    CH = 128
    n_ch = b_per_w // CH
    assert b_per_w % CH == 0
    KD = D // _L

    mesh = plsc.VectorSubcoreMesh(core_axis_name="c", subcore_axis_name="s")

    @functools.partial(
        pl.kernel,
        mesh=mesh,
        out_type=jax.ShapeDtypeStruct((B, D), jnp.float32),
        scratch_types=[
            pltpu.VMEM((b_per_w,), jnp.int32),
            pltpu.VMEM((b_per_w,), jnp.int32),
            pltpu.VMEM((b_per_w,), jnp.int32),
            pltpu.VMEM((CH, W), jnp.float32),
            pltpu.VMEM((CH, W), jnp.float32),
            pltpu.VMEM((CH, D), jnp.float32),
            pltpu.SemaphoreType.DMA,
            pltpu.SemaphoreType.DMA,
        ],
        compiler_params=pltpu.CompilerParams(needs_layout_passes=False),
    )
    def gather(x2_hbm, idx_hbm, out_hbm, idx_v, pair_v, half_v,
               buf_a, buf_b, out_v, sem_a, sem_b):
        wid = lax.axis_index("s") * NC + lax.axis_index("c")
        base = wid * b_per_w
        pltpu.sync_copy(idx_hbm.at[pl.ds(base, b_per_w)], idx_v)

        @pl.loop(0, b_per_w // _L)
        def _split(i):
            v = idx_v[pl.ds(i * _L, _L)]
            pair_v[pl.ds(i * _L, _L)] = lax.shift_right_logical(v, 1)
            half_v[pl.ds(i * _L, _L)] = lax.bitwise_and(v, 1)

        bufs = (buf_a, buf_b)
        sems = (sem_a, sem_b)
        lanes = lax.iota(jnp.int32, _L)

        def start(c, slot):
            pltpu.async_copy(
                x2_hbm.at[pair_v.at[pl.ds(c * CH, CH)]], bufs[slot],
                sems[slot])

        def wait(slot):
            pltpu.make_async_copy(
                x2_hbm.at[pl.ds(0, CH)], bufs[slot], sems[slot]).wait()

        def select_and_write(c, slot):
            buf = bufs[slot]

            @pl.loop(0, CH)
            def _row(j):
                jv = jnp.zeros((_L,), jnp.int32) + j
                hv = plsc.load_gather(half_v, [jv + c * CH])
                col0 = hv * D
                for k in range(KD):
                    cv = col0 + lanes + (k * _L)
                    val = plsc.load_gather(buf, [jv, cv])
                    out_v[j, pl.ds(k * _L, _L)] = val

            pltpu.sync_copy(out_v, out_hbm.at[pl.ds(base + c * CH, CH)])

        start(0, 0)
        for c in range(n_ch):
            slot = c % 2
            if c + 1 < n_ch:
                start(c + 1, (c + 1) % 2)
            wait(slot)
            select_and_write(c, slot)

    return gather


def kernel(x, selected_rows):
    V, D = x.shape
    B = selected_rows.shape[0]
    x2 = x.reshape(V // 2, 2 * D)
    return _build_gather(V, D, B)(x2, selected_rows.astype(jnp.int32))


# speedup vs baseline: 1.3406x; 1.3406x over previous
import functools

import jax
import jax.numpy as jnp
from jax import lax
from jax.experimental import pallas as pl
from jax.experimental.pallas import tpu as pltpu
from jax.experimental.pallas import tpu_sc as plsc

_L = 16


@functools.lru_cache(maxsize=None)
def _build_gather(V: int, D: int, B: int):
    info = plsc.get_sparse_core_info()
    NC, NS = info.num_cores, info.num_subcores
    NW = NC * NS
    W = 2 * D
    assert B % NW == 0 and D % _L == 0 and V % 2 == 0
    b_per_w = B // NW
    CH = 128
    n_ch = b_per_w // CH
    assert b_per_w % CH == 0
    KD = D // _L

    mesh = plsc.VectorSubcoreMesh(core_axis_name="c", subcore_axis_name="s")

    @functools.partial(
        pl.kernel,
        mesh=mesh,
        out_type=jax.ShapeDtypeStruct((B, D), jnp.float32),
        scratch_types=[
            pltpu.VMEM((b_per_w,), jnp.int32),
            pltpu.VMEM((b_per_w,), jnp.int32),
            pltpu.VMEM((b_per_w,), jnp.int32),
            pltpu.VMEM((CH, W), jnp.float32),
            pltpu.VMEM((CH, W), jnp.float32),
            pltpu.VMEM((CH, D), jnp.float32),
            pltpu.SemaphoreType.DMA,
            pltpu.SemaphoreType.DMA,
        ],
        compiler_params=pltpu.CompilerParams(needs_layout_passes=False),
    )
    def gather(x_hbm, x2_hbm, idx_hbm, out_hbm, idx_v, pair_v, half_v,
               buf_a, buf_b, out_v, sem_a, sem_b):
        del x_hbm
        wid = lax.axis_index("s") * NC + lax.axis_index("c")
        base = wid * b_per_w
        pltpu.sync_copy(idx_hbm.at[pl.ds(base, b_per_w)], idx_v)

        @pl.loop(0, b_per_w // _L)
        def _split(i):
            v = idx_v[pl.ds(i * _L, _L)]
            pair_v[pl.ds(i * _L, _L)] = lax.shift_right_logical(v, 1)
            half_v[pl.ds(i * _L, _L)] = lax.bitwise_and(v, 1)

        bufs = (buf_a, buf_b)
        sems = (sem_a, sem_b)
        lanes = lax.iota(jnp.int32, _L)

        def start(c, slot):
            pltpu.async_copy(
                x2_hbm.at[pair_v.at[pl.ds(c * CH, CH)]], bufs[slot],
                sems[slot])

        def wait(slot):
            pltpu.make_async_copy(
                x2_hbm.at[pl.ds(0, CH)], bufs[slot], sems[slot]).wait()

        def select_and_write(c, slot):
            buf = bufs[slot]

            @pl.loop(0, CH)
            def _row(j):
                jv = jnp.zeros((_L,), jnp.int32) + j
                hv = plsc.load_gather(half_v, [jv + c * CH])
                col0 = hv * D
                for k in range(KD):
                    cv = col0 + lanes + (k * _L)
                    val = plsc.load_gather(buf, [jv, cv])
                    out_v[j, pl.ds(k * _L, _L)] = val

            pltpu.sync_copy(out_v, out_hbm.at[pl.ds(base + c * CH, CH)])

        start(0, 0)
        for c in range(n_ch):
            slot = c % 2
            if c + 1 < n_ch:
                start(c + 1, (c + 1) % 2)
            wait(slot)
            select_and_write(c, slot)

    return gather


def kernel(x, selected_rows):
    V, D = x.shape
    B = selected_rows.shape[0]
    x2 = x.reshape(V // 2, 2 * D)
    return _build_gather(V, D, B)(x, x2, selected_rows.astype(jnp.int32))


# speedup vs baseline: 2.3270x; 1.7358x over previous
import functools

import jax
import jax.numpy as jnp
from jax import lax
from jax.experimental import pallas as pl
from jax.experimental.pallas import tpu as pltpu
from jax.experimental.pallas import tpu_sc as plsc

_L = 16


@functools.lru_cache(maxsize=None)
def _build_gather(V: int, D: int, B: int):
    info = plsc.get_sparse_core_info()
    NC, NS = info.num_cores, info.num_subcores
    NW = NC * NS
    assert B % NW == 0 and D % _L == 0 and V % 8 == 0
    b_per_w = B // NW
    CH = 32
    n_ch = b_per_w // CH
    assert b_per_w % CH == 0
    KD = D // _L

    mesh = plsc.VectorSubcoreMesh(core_axis_name="c", subcore_axis_name="s")

    @functools.partial(
        pl.kernel,
        mesh=mesh,
        out_type=jax.ShapeDtypeStruct((B, D), jnp.float32),
        scratch_types=[
            pltpu.VMEM((b_per_w,), jnp.int32),
            pltpu.VMEM((b_per_w,), jnp.int32),
            pltpu.VMEM((b_per_w,), jnp.int32),
            pltpu.VMEM((CH, 8, D), jnp.float32),
            pltpu.VMEM((CH, 8, D), jnp.float32),
            pltpu.VMEM((CH, D), jnp.float32),
            pltpu.SemaphoreType.DMA,
            pltpu.SemaphoreType.DMA,
        ],
        compiler_params=pltpu.CompilerParams(needs_layout_passes=False),
    )
    def gather(x3_hbm, idx_hbm, out_hbm, idx_v, grp_v, sub_v,
               buf_a, buf_b, out_v, sem_a, sem_b):
        wid = lax.axis_index("s") * NC + lax.axis_index("c")
        base = wid * b_per_w
        pltpu.sync_copy(idx_hbm.at[pl.ds(base, b_per_w)], idx_v)

        @pl.loop(0, b_per_w // _L)
        def _split(i):
            v = idx_v[pl.ds(i * _L, _L)]
            grp_v[pl.ds(i * _L, _L)] = lax.shift_right_logical(v, 3)
            sub_v[pl.ds(i * _L, _L)] = lax.bitwise_and(v, 7)

        bufs = (buf_a, buf_b)
        sems = (sem_a, sem_b)
        lanes = lax.iota(jnp.int32, _L)

        def fire(c, slot):
            buf, sem = bufs[slot], sems[slot]

            @pl.loop(0, CH // _L)
            def _grp16(g):
                vec = grp_v[pl.ds(c * CH + g * _L, _L)]
                for l in range(_L):
                    r = jnp.sum(jnp.where(lanes == l, vec, 0))
                    pltpu.async_copy(x3_hbm.at[r], buf.at[g * _L + l], sem)

        def wait(slot):
            pltpu.make_async_copy(
                x3_hbm.at[pl.ds(0, CH)], bufs[slot], sems[slot]).wait()

        def select(c, slot):
            buf = bufs[slot]

            @pl.loop(0, CH)
            def _row(j):
                jv = jnp.zeros((_L,), jnp.int32) + j
                sv = plsc.load_gather(sub_v, [jv + c * CH])
                for k in range(KD):
                    cv = lanes + (k * _L)
                    val = plsc.load_gather(buf, [jv, sv, cv])
                    out_v[j, pl.ds(k * _L, _L)] = val

        fire(0, 0)
        for c in range(n_ch):
            slot = c % 2
            if c + 1 < n_ch:
                fire(c + 1, (c + 1) % 2)
            wait(slot)
            select(c, slot)
            pltpu.sync_copy(out_v, out_hbm.at[pl.ds(base + c * CH, CH)])

    return gather


def kernel(x, selected_rows):
    V, D = x.shape
    B = selected_rows.shape[0]
    x3 = x.reshape(V // 8, 8, D)
    return _build_gather(V, D, B)(x3, selected_rows.astype(jnp.int32))
